# baseline (speedup 1.0000x reference)
import functools

import jax
import jax.numpy as jnp
from jax import lax
from jax.experimental import pallas as pl
from jax.experimental.pallas import tpu as pltpu

_B, _C, _H, _W, _E = 2, 384, 48, 48, 128
_N = _H * _W
_SIGMA = 0.05
_SCALE = float(_E) ** (-0.5)
_NEG = float(jnp.finfo(jnp.float32).min)
_LOG2E = 1.4426950408889634
_CB = 200.0 * _LOG2E / ((_H - 1) * (_H - 1))

_BLK_KV = 768
_BLK_Q = 256
_BLK_K = 256
_NW = 32
_ROWS_PER_W = (_B * _N) // _NW


def _sc_gather(table, idx):
    from jax.experimental.pallas import tpu_sc as plsc
    mesh = plsc.VectorSubcoreMesh(core_axis_name="c", subcore_axis_name="s")

    @functools.partial(
        pl.kernel, mesh=mesh,
        out_type=jax.ShapeDtypeStruct((_B * _N, _C), jnp.float32),
        scratch_types=[
            pltpu.VMEM((_ROWS_PER_W,), jnp.int32),
            pltpu.VMEM((_ROWS_PER_W, _C), jnp.float32),
            pltpu.SemaphoreType.DMA,
        ],
    )
    def k(table_hbm, idx_hbm, out_hbm, idx_v, rows_v, sem):
        wid = lax.axis_index("s") * 2 + lax.axis_index("c")
        base = wid * _ROWS_PER_W
        pltpu.sync_copy(idx_hbm.at[pl.ds(base, _ROWS_PER_W)], idx_v)
        pltpu.async_copy(table_hbm.at[idx_v], rows_v, sem).wait()
        pltpu.sync_copy(rows_v, out_hbm.at[pl.ds(base, _ROWS_PER_W)])

    return k(table, idx)


def _kv_kernel(x_ref, w_ref, o_ref):
    o_ref[0] = jax.lax.dot_general(
        x_ref[0].astype(jnp.bfloat16), w_ref[...].astype(jnp.bfloat16),
        (((1,), (1,)), ((), ())),
        preferred_element_type=jnp.float32,
    )


def _attn_kernel(nkb_ref, x_ref, wq_ref, k_ref, v_ref, qi_ref, ki_ref,
                 col_ref, qcol_ref, hi_ref, wp_ref, o_ref):
    b = pl.program_id(0)
    q = (jax.lax.dot_general(
        wq_ref[...].astype(jnp.bfloat16), x_ref[0].astype(jnp.bfloat16),
        (((1,), (0,)), ((), ())),
        preferred_element_type=jnp.float32,
    ) * (_SCALE * _LOG2E)).astype(jnp.bfloat16)
    qi = qi_ref[...]
    qcol = qcol_ref[...]

    def body(kb, carry):
        acc, l = carry
        off = kb * _BLK_K
        k_blk = k_ref[0, pl.ds(off, _BLK_K), :].astype(jnp.bfloat16)
        v_blk = v_ref[0, pl.ds(off, _BLK_K), :].astype(jnp.bfloat16)
        s = jax.lax.dot_general(
            q, k_blk, (((0,), (1,)), ((), ())),
            preferred_element_type=jnp.float32,
        )
        cross = jax.lax.dot_general(
            qi, ki_ref[0, :, pl.ds(off, _BLK_K)], (((0,), (0,)), ((), ())),
            preferred_element_type=jnp.float32,
        )
        s = (s + qcol) + (cross * (2.0 * _CB) + col_ref[0, :, pl.ds(off, _BLK_K)])
        p = jnp.exp2(s)
        l = l + jnp.sum(p, axis=1, keepdims=True)
        acc = acc + jax.lax.dot_general(
            p.astype(jnp.bfloat16), v_blk, (((1,), (0,)), ((), ())),
            preferred_element_type=jnp.float32,
        )
        return acc, l

    acc, l = lax.fori_loop(
        0, nkb_ref[b], body,
        (jnp.zeros((_BLK_Q, _E), jnp.float32),
         jnp.zeros((_BLK_Q, 1), jnp.float32)))
    acc = (acc * (1.0 / jnp.maximum(l, 1e-30))).astype(jnp.bfloat16)
    delta_t = jax.lax.dot_general(
        wp_ref[...].astype(jnp.bfloat16), acc, (((1,), (1,)), ((), ())),
        preferred_element_type=jnp.float32,
    )
    o_ref[0] = x_ref[0] + jnp.where(hi_ref[0] > 0, delta_t, 0.0)


@jax.jit
def kernel(feat, mask_hi, Wq, Wk, Wv, Wp):
    x = feat.reshape(_B, _C, _N)
    wkv = jnp.concatenate([Wk, Wv], axis=0)
    hi_b = mask_hi.reshape(_B, _N)
    hi = hi_b.reshape(_B, 1, _N).astype(jnp.float32)

    cnt_lo = _N - jnp.sum(hi_b.astype(jnp.int32), axis=1)
    perm_lo = jnp.argsort(hi_b, axis=1, stable=True).astype(jnp.int32)
    nkb = (cnt_lo + _BLK_K - 1) // _BLK_K

    idx = jnp.arange(_N, dtype=jnp.int32)
    gi = (idx // _W).astype(jnp.float32)
    gj = (idx % _W).astype(jnp.float32)
    zero = jnp.zeros((_N,), jnp.float32)
    qcoords = jnp.stack([gi, gj, zero, zero, zero, zero, zero, zero],
                        axis=0).astype(jnp.bfloat16)
    kgi = gi[perm_lo]
    kgj = gj[perm_lo]
    zb = jnp.zeros((_B, _N), jnp.float32)
    kcoords = jnp.stack([kgi, kgj, zb, zb, zb, zb, zb, zb],
                        axis=1).astype(jnp.bfloat16)
    col = jnp.where(idx[None, :] < cnt_lo[:, None],
                    -_CB * (kgi * kgi + kgj * kgj),
                    _NEG).reshape(_B, 1, _N)
    qcol = (-_CB * (gi * gi + gj * gj)).reshape(_N, 1)

    ff_tm = x.transpose(0, 2, 1).reshape(_B * _N, _C)
    idx_flat = (perm_lo + (jnp.arange(_B, dtype=jnp.int32) * _N)[:, None]
                ).reshape(_B * _N)
    ffl = _sc_gather(ff_tm, idx_flat).reshape(_B, _N, _C)

    kvc = pl.pallas_call(
        _kv_kernel,
        grid=(_B, _N // _BLK_KV),
        in_specs=[
            pl.BlockSpec((1, _BLK_KV, _C), lambda b, n: (b, n, 0)),
            pl.BlockSpec((2 * _E, _C), lambda b, n: (0, 0)),
        ],
        out_specs=pl.BlockSpec((1, _BLK_KV, 2 * _E), lambda b, n: (b, n, 0)),
        out_shape=jax.ShapeDtypeStruct((_B, _N, 2 * _E), jnp.float32),
    )(ffl, wkv)

    grid_spec = pltpu.PrefetchScalarGridSpec(
        num_scalar_prefetch=1,
        grid=(_B, _N // _BLK_Q),
        in_specs=[
            pl.BlockSpec((1, _C, _BLK_Q), lambda b, q, s: (b, 0, q)),
            pl.BlockSpec((_E, _C), lambda b, q, s: (0, 0)),
            pl.BlockSpec((1, _N, _E), lambda b, q, s: (b, 0, 0)),
            pl.BlockSpec((1, _N, _E), lambda b, q, s: (b, 0, 1)),
            pl.BlockSpec((8, _BLK_Q), lambda b, q, s: (0, q)),
            pl.BlockSpec((1, 8, _N), lambda b, q, s: (b, 0, 0)),
            pl.BlockSpec((1, 1, _N), lambda b, q, s: (b, 0, 0)),
            pl.BlockSpec((_BLK_Q, 1), lambda b, q, s: (q, 0)),
            pl.BlockSpec((1, 1, _BLK_Q), lambda b, q, s: (b, 0, q)),
            pl.BlockSpec((_C, _E), lambda b, q, s: (0, 0)),
        ],
        out_specs=pl.BlockSpec((1, _C, _BLK_Q), lambda b, q, s: (b, 0, q)),
    )

    out = pl.pallas_call(
        _attn_kernel,
        grid_spec=grid_spec,
        out_shape=jax.ShapeDtypeStruct((_B, _C, _N), jnp.float32),
    )(nkb, x, Wq, kvc, kvc, qcoords, kcoords, col, qcol, hi, Wp)

    return out.reshape(_B, _C, _H, _W)
